# Initial kernel scaffold; baseline (speedup 1.0000x reference)
import jax
import jax.numpy as jnp
from jax.experimental import pallas as pl


def kernel(event, table):
    raise NotImplementedError("write your pallas kernel here")


# speedup vs baseline: 3.9510x; 3.9510x over previous
import functools

import jax
import jax.numpy as jnp
from jax import lax
from jax.experimental import pallas as pl
from jax.experimental.pallas import tpu as pltpu
from jax.experimental.pallas import tpu_sc as plsc

_NC = 2
_NS = 16
_NW = _NC * _NS
_CHUNK = 512
_SUB = 128


@functools.cache
def _build(B, D):
    b_per_w = B // _NW
    n_chunks = b_per_w // _CHUNK
    mesh = plsc.VectorSubcoreMesh(core_axis_name="c", subcore_axis_name="s")

    @functools.partial(
        pl.kernel,
        out_type=jax.ShapeDtypeStruct((B, D), jnp.float32),
        mesh=mesh,
        scratch_types=[
            pltpu.VMEM((_CHUNK,), jnp.int32),
            pltpu.VMEM((_CHUNK, D), jnp.float32),
            pltpu.SemaphoreType.DMA,
        ],
        compiler_params=pltpu.CompilerParams(use_tc_tiling_on_sc=False),
    )
    def gather_kernel(table_hbm, idx_hbm, out_hbm, idx_v, rows_v, sem):
        wid = lax.axis_index("s") * _NC + lax.axis_index("c")
        base = wid * b_per_w

        def body(g, carry):
            off = pl.multiple_of(base + g * _CHUNK, _CHUNK)
            pltpu.sync_copy(idx_hbm.at[pl.ds(off, _CHUNK)], idx_v)
            copies = []
            for j in range(_CHUNK // _SUB):
                copies.append(
                    pltpu.async_copy(
                        table_hbm.at[idx_v.at[pl.ds(j * _SUB, _SUB)]],
                        rows_v.at[pl.ds(j * _SUB, _SUB), :],
                        sem,
                    )
                )
            for c in copies:
                c.wait()
            pltpu.sync_copy(rows_v, out_hbm.at[pl.ds(off, _CHUNK), :])
            return carry

        lax.fori_loop(0, n_chunks, body, 0)

    return gather_kernel


def kernel(event, table):
    B = event.size
    D = table.shape[1]
    flat = event.reshape(B)
    out = _build(B, D)(table, flat)
    return out.reshape(event.shape + (D,))


# speedup vs baseline: 4.2127x; 1.0663x over previous
import functools

import jax
import jax.numpy as jnp
from jax import lax
from jax.experimental import pallas as pl
from jax.experimental.pallas import tpu as pltpu
from jax.experimental.pallas import tpu_sc as plsc

_NC = 2
_NS = 16
_NW = _NC * _NS
_CHUNK = 512
_SUB = 128


@functools.cache
def _build(B, D):
    b_per_w = B // _NW
    n_chunks = b_per_w // _CHUNK
    n_pairs = n_chunks // 2
    mesh = plsc.VectorSubcoreMesh(core_axis_name="c", subcore_axis_name="s")

    @functools.partial(
        pl.kernel,
        out_type=jax.ShapeDtypeStruct((B, D), jnp.float32),
        mesh=mesh,
        scratch_types=[
            pltpu.VMEM((b_per_w,), jnp.int32),
            pltpu.VMEM((_CHUNK, D), jnp.float32),
            pltpu.VMEM((_CHUNK, D), jnp.float32),
            pltpu.SemaphoreType.DMA,
            pltpu.SemaphoreType.DMA,
            pltpu.SemaphoreType.DMA,
            pltpu.SemaphoreType.DMA,
        ],
        compiler_params=pltpu.CompilerParams(use_tc_tiling_on_sc=False),
    )
    def gather_kernel(table_hbm, idx_hbm, out_hbm,
                      idx_v, buf0, buf1, gsem0, gsem1, wsem0, wsem1):
        wid = lax.axis_index("s") * _NC + lax.axis_index("c")
        base = pl.multiple_of(wid * b_per_w, _CHUNK)
        pltpu.sync_copy(idx_hbm.at[pl.ds(base, b_per_w)], idx_v)

        def fire_gather(c, buf, sem):
            for j in range(_CHUNK // _SUB):
                off = pl.multiple_of(c * _CHUNK + j * _SUB, _SUB)
                pltpu.async_copy(
                    table_hbm.at[idx_v.at[pl.ds(off, _SUB)]],
                    buf.at[pl.ds(j * _SUB, _SUB), :],
                    sem,
                )

        def wait_gather(buf, sem):
            pltpu.make_async_copy(out_hbm.at[pl.ds(0, _CHUNK), :], buf, sem).wait()

        def fire_write(buf, c, sem):
            off = pl.multiple_of(base + c * _CHUNK, _CHUNK)
            pltpu.async_copy(buf, out_hbm.at[pl.ds(off, _CHUNK), :], sem)

        def wait_write(buf, sem):
            pltpu.make_async_copy(buf, out_hbm.at[pl.ds(0, _CHUNK), :], sem).wait()

        fire_gather(0, buf0, gsem0)
        fire_gather(1, buf1, gsem1)

        def body(i, carry):
            wait_gather(buf0, gsem0)
            fire_write(buf0, 2 * i, wsem0)
            wait_gather(buf1, gsem1)
            fire_write(buf1, 2 * i + 1, wsem1)

            @pl.when(i < n_pairs - 1)
            def _refill():
                wait_write(buf0, wsem0)
                fire_gather(2 * i + 2, buf0, gsem0)
                wait_write(buf1, wsem1)
                fire_gather(2 * i + 3, buf1, gsem1)

            return carry

        lax.fori_loop(0, n_pairs, body, 0)
        wait_write(buf0, wsem0)
        wait_write(buf1, wsem1)

    return gather_kernel


def kernel(event, table):
    B = event.size
    D = table.shape[1]
    flat = event.reshape(B)
    out = _build(B, D)(table, flat)
    return out.reshape(event.shape + (D,))
